# baseline (speedup 1.0000x reference)
import jax
import jax.numpy as jnp
from jax.experimental import pallas as pl
from jax.experimental.pallas import tpu as pltpu

_N = 100000
_D = 128
_R = 5000
_NBLK = _N // _R


def _stage1(z_ref, zp_ref, zn_ref, x_ref, xh_ref, pdot_ref, ndot_ref, acc_ref):
    i = pl.program_id(0)

    @pl.when(i == 0)
    def _init():
        acc_ref[0] = 0.0

    z = z_ref[...]
    ones_col = jnp.ones((_D, 1), dtype=jnp.float32)
    dnums = (((1,), (0,)), ((), ()))
    pdot_ref[...] = jax.lax.dot_general(z * zp_ref[...], ones_col, dnums,
                                        preferred_element_type=jnp.float32)
    ndot_ref[...] = jax.lax.dot_general(z * zn_ref[...], ones_col, dnums,
                                        preferred_element_type=jnp.float32)
    diff = x_ref[...] - xh_ref[...]
    acc_ref[0] += jnp.sum(diff * diff)


def _stage2(pdot_ref, ndot_ref, out_ref):
    pos = jnp.sum(jax.nn.log_sigmoid(pdot_ref[...]))
    neg = jnp.sum(jax.nn.log_sigmoid(-ndot_ref[...]))
    out_ref[0] = pos
    out_ref[1] = neg


def kernel(out, x_full, xhat_full, lamb):
    row_spec = pl.BlockSpec((_R, _D), lambda i: (i, 0))
    col_spec = pl.BlockSpec((_R, 1), lambda i: (i, 0))
    pdot, ndot, mse_sum = pl.pallas_call(
        _stage1,
        grid=(_NBLK,),
        in_specs=[
            pl.BlockSpec((_R, _D), lambda i: (i, 0)),
            pl.BlockSpec((_R, _D), lambda i: (i + _NBLK, 0)),
            pl.BlockSpec((_R, _D), lambda i: (i + 2 * _NBLK, 0)),
            row_spec,
            row_spec,
        ],
        out_specs=[col_spec, col_spec,
                   pl.BlockSpec(memory_space=pltpu.SMEM)],
        out_shape=[
            jax.ShapeDtypeStruct((_N, 1), jnp.float32),
            jax.ShapeDtypeStruct((_N, 1), jnp.float32),
            jax.ShapeDtypeStruct((1,), jnp.float32),
        ],
    )(out, out, out, x_full, xhat_full)

    sums = pl.pallas_call(
        _stage2,
        out_specs=pl.BlockSpec(memory_space=pltpu.SMEM),
        out_shape=jax.ShapeDtypeStruct((2,), jnp.float32),
    )(pdot.reshape(_N), ndot.reshape(_N))

    lamb = jnp.clip(lamb, 1e-08, 1.0 - 1e-08)
    pos_loss = sums[0] / _N
    neg_loss = sums[1] / _N
    mse = mse_sum[0] / (_N * _D)
    return lamb * mse + (1.0 - lamb) * (-pos_loss - neg_loss)


# speedup vs baseline: 1.8365x; 1.8365x over previous
import jax
import jax.numpy as jnp
from jax.experimental import pallas as pl
from jax.experimental.pallas import tpu as pltpu

_N = 100000
_D = 128
_R = 10000
_NBLK = _N // _R


def _body(z_ref, zp_ref, zn_ref, x_ref, xh_ref, acc_ref):
    i = pl.program_id(0)

    @pl.when(i == 0)
    def _init():
        acc_ref[0] = 0.0
        acc_ref[1] = 0.0
        acc_ref[2] = 0.0

    z = z_ref[...]
    ones_row = jnp.ones((1, _D), dtype=jnp.float32)
    dnums = (((1,), (1,)), ((), ()))
    pdot = jax.lax.dot_general(ones_row, z * zp_ref[...], dnums,
                               preferred_element_type=jnp.float32)
    ndot = jax.lax.dot_general(ones_row, z * zn_ref[...], dnums,
                               preferred_element_type=jnp.float32)
    pos_part = jnp.sum(jax.nn.log_sigmoid(pdot))
    neg_part = jnp.sum(jax.nn.log_sigmoid(-ndot))
    diff = x_ref[...] - xh_ref[...]
    mse_part = jnp.sum(diff * diff)
    acc_ref[0] += pos_part
    acc_ref[1] += neg_part
    acc_ref[2] += mse_part


def kernel(out, x_full, xhat_full, lamb):
    row_spec = pl.BlockSpec((_R, _D), lambda i: (i, 0))
    sums = pl.pallas_call(
        _body,
        grid=(_NBLK,),
        in_specs=[
            pl.BlockSpec((_R, _D), lambda i: (i, 0)),
            pl.BlockSpec((_R, _D), lambda i: (i + _NBLK, 0)),
            pl.BlockSpec((_R, _D), lambda i: (i + 2 * _NBLK, 0)),
            row_spec,
            row_spec,
        ],
        out_specs=pl.BlockSpec(memory_space=pltpu.SMEM),
        out_shape=jax.ShapeDtypeStruct((3,), jnp.float32),
    )(out, out, out, x_full, xhat_full)

    lamb = jnp.clip(lamb, 1e-08, 1.0 - 1e-08)
    pos_loss = sums[0] / _N
    neg_loss = sums[1] / _N
    mse = sums[2] / (_N * _D)
    return lamb * mse + (1.0 - lamb) * (-pos_loss - neg_loss)
